# baseline (speedup 1.0000x reference)
import functools

import jax
import jax.numpy as jnp
import numpy as np
from jax import lax
from jax.experimental import pallas as pl
from jax.experimental.pallas import tpu as pltpu
from jax.experimental.pallas import tpu_sc as plsc

N_ENT = 1000000
N_REL = 200
DIM = 64
MAX_SCALE = 2.5
MARGIN = 8.0
B = 4096

_NC = 1
_NS = 16
_NW = _NC * _NS
_BPW = B // _NW


@functools.cache
def _build_sc_gather():
    mesh = plsc.VectorSubcoreMesh(
        core_axis_name="c", subcore_axis_name="s", num_cores=_NC)

    @functools.partial(
        pl.kernel,
        mesh=mesh,
        out_type=[
            jax.ShapeDtypeStruct((B,), jnp.float32),
            jax.ShapeDtypeStruct((B,), jnp.float32),
        ],
        scratch_types=[
            pltpu.VMEM((_BPW,), jnp.int32),
            pltpu.VMEM((_BPW,), jnp.int32),
            pltpu.VMEM((_BPW,), jnp.float32),
            pltpu.VMEM((_BPW,), jnp.float32),
            pltpu.SemaphoreType.DMA,
        ],
    )
    def sc_gather(u_hbm, v_hbm, bh_hbm, bt_hbm,
                  bh_out, bt_out,
                  uidx_v, vidx_v, bh_v, bt_v, sem):
        wid = lax.axis_index("s") * _NC + lax.axis_index("c")
        base = wid * _BPW
        pltpu.sync_copy(u_hbm.at[pl.ds(base, _BPW)], uidx_v)
        pltpu.sync_copy(v_hbm.at[pl.ds(base, _BPW)], vidx_v)
        c1 = pltpu.async_copy(bh_hbm.at[uidx_v], bh_v, sem)
        c2 = pltpu.async_copy(bt_hbm.at[vidx_v], bt_v, sem)
        c1.wait()
        c2.wait()
        pltpu.sync_copy(bh_v, bh_out.at[pl.ds(base, _BPW)])
        pltpu.sync_copy(bt_v, bt_out.at[pl.ds(base, _BPW)])

    return sc_gather


def _pair_consts():
    P = np.zeros((DIM, DIM), np.float32)
    E = np.zeros((DIM, DIM), np.float32)
    O = np.zeros((DIM, DIM), np.float32)
    for k in range(DIM // 2):
        P[2 * k + 1, 2 * k] = -1.0
        P[2 * k, 2 * k + 1] = 1.0
        E[2 * k, 2 * k] = 1.0
        E[2 * k, 2 * k + 1] = 1.0
        O[2 * k + 1, 2 * k] = 1.0
        O[2 * k + 1, 2 * k + 1] = 1.0
    return P, E, O


_P_MAT, _E_MAT, _O_MAT = _pair_consts()

_BM = 512


def _tc_body(u_sref, v_sref, scale_ref, emb_ref, r_ref,
             diag_ref, rbias_ref, bh_ref, bt_ref,
             pm_ref, em_ref, om_ref, o_ref,
             h_all, t_all, sem):
    i = pl.program_id(0)

    @pl.when(i == 0)
    def _gather():
        def issue(k, _):
            pltpu.make_async_copy(
                emb_ref.at[pl.ds(u_sref[k], 1), :],
                h_all.at[pl.ds(k, 1), :], sem).start()
            pltpu.make_async_copy(
                emb_ref.at[pl.ds(v_sref[k], 1), :],
                t_all.at[pl.ds(k, 1), :], sem).start()
            return 0
        lax.fori_loop(0, B, issue, 0, unroll=16)

        pltpu.make_async_copy(
            emb_ref.at[pl.ds(0, B), :], h_all, sem).wait()
        pltpu.make_async_copy(
            emb_ref.at[pl.ds(0, B), :], t_all, sem).wait()

    scale = scale_ref[0, 0]
    dot = functools.partial(
        lax.dot_general,
        dimension_numbers=(((1,), (0,)), ((), ())),
        preferred_element_type=jnp.float32,
    )

    h = h_all[pl.ds(i * _BM, _BM), :]
    t = t_all[...]

    rel = lax.broadcasted_iota(jnp.int32, (_BM, N_REL), 1)
    onehot = (rel == r_ref[...]).astype(jnp.float32)
    rd = dot(onehot, diag_ref[...])
    rb = dot(onehot, rbias_ref[...])

    a_bc = dot(rd, em_ref[...])
    b_bc = dot(rd, om_ref[...])
    inv_nrm = 1.0 / jnp.maximum(jnp.sqrt(a_bc * a_bc + b_bc * b_bc), 1e-15)
    h_sw = dot(h, pm_ref[...])
    x_rot = (a_bc * h + b_bc * h_sw) * inv_nrm

    col = lax.broadcasted_iota(jnp.int32, (_BM, DIM), 1)
    time = jax.nn.sigmoid(x_rot[:, 0:1]) * scale + 1.1
    x = x_rot + rb
    xn = jnp.where(col > 0, x, 0.0)
    s2 = jnp.sum(xn * xn, axis=1, keepdims=True)
    factor = jnp.sqrt((time * time - 1.0) / s2)
    h_l = jnp.where(col == 0, -time, x * factor)

    scores = lax.dot_general(
        h_l, t,
        dimension_numbers=(((1,), (1,)), ((), ())),
        preferred_element_type=jnp.float32,
    )
    o_ref[...] = MARGIN + 2.0 * scores + bh_ref[...] + bt_ref[...]


def kernel(u_idx, r_idx, v_idx, emb_entity, relation_bias, diag,
           bias_head, bias_tail, scale):
    u_idx = u_idx.astype(jnp.int32)
    v_idx = v_idx.astype(jnp.int32)
    r_idx = r_idx.astype(jnp.int32)

    bh_g, bt_g = _build_sc_gather()(u_idx, v_idx, bias_head, bias_tail)

    scale2 = scale.reshape(1, 1).astype(jnp.float32)
    r_col = r_idx.reshape(B, 1)
    bh_col = bh_g.reshape(B, 1)
    bt_row = bt_g.reshape(1, B)

    out = pl.pallas_call(
        _tc_body,
        grid=(B // _BM,),
        in_specs=[
            pl.BlockSpec(memory_space=pltpu.SMEM),
            pl.BlockSpec(memory_space=pltpu.SMEM),
            pl.BlockSpec((1, 1), lambda i: (0, 0), memory_space=pltpu.SMEM),
            pl.BlockSpec(memory_space=pl.ANY),
            pl.BlockSpec((_BM, 1), lambda i: (i, 0)),
            pl.BlockSpec((N_REL, DIM), lambda i: (0, 0)),
            pl.BlockSpec((N_REL, DIM), lambda i: (0, 0)),
            pl.BlockSpec((_BM, 1), lambda i: (i, 0)),
            pl.BlockSpec((1, B), lambda i: (0, 0)),
            pl.BlockSpec((DIM, DIM), lambda i: (0, 0)),
            pl.BlockSpec((DIM, DIM), lambda i: (0, 0)),
            pl.BlockSpec((DIM, DIM), lambda i: (0, 0)),
        ],
        out_specs=pl.BlockSpec((_BM, B), lambda i: (i, 0)),
        out_shape=jax.ShapeDtypeStruct((B, B), jnp.float32),
        scratch_shapes=[
            pltpu.VMEM((B, DIM), jnp.float32),
            pltpu.VMEM((B, DIM), jnp.float32),
            pltpu.SemaphoreType.DMA,
        ],
        compiler_params=pltpu.CompilerParams(
            dimension_semantics=("arbitrary",),
            disable_bounds_checks=True,
        ),
    )(u_idx, v_idx, scale2, emb_entity, r_col,
      diag, relation_bias, bh_col, bt_row,
      jnp.asarray(_P_MAT), jnp.asarray(_E_MAT), jnp.asarray(_O_MAT))
    return out


# speedup vs baseline: 1.0720x; 1.0720x over previous
import functools

import jax
import jax.numpy as jnp
import numpy as np
from jax import lax
from jax.experimental import pallas as pl
from jax.experimental.pallas import tpu as pltpu
from jax.experimental.pallas import tpu_sc as plsc

N_ENT = 1000000
N_REL = 200
DIM = 64
MAX_SCALE = 2.5
MARGIN = 8.0
B = 4096

_NC = 2
_NS = 16
_NW = _NC * _NS
_BPW = B // _NW


@functools.cache
def _build_sc_gather():
    mesh = plsc.VectorSubcoreMesh(
        core_axis_name="c", subcore_axis_name="s", num_cores=_NC)

    @functools.partial(
        pl.kernel,
        mesh=mesh,
        out_type=[
            jax.ShapeDtypeStruct((B, DIM), jnp.float32),
            jax.ShapeDtypeStruct((B, DIM), jnp.float32),
            jax.ShapeDtypeStruct((B,), jnp.float32),
            jax.ShapeDtypeStruct((B,), jnp.float32),
        ],
        scratch_types=[
            pltpu.VMEM((_BPW,), jnp.int32),
            pltpu.VMEM((_BPW,), jnp.int32),
            pltpu.VMEM((_BPW, DIM), jnp.float32),
            pltpu.VMEM((_BPW, DIM), jnp.float32),
            pltpu.VMEM((_BPW,), jnp.float32),
            pltpu.VMEM((_BPW,), jnp.float32),
            pltpu.SemaphoreType.DMA,
            pltpu.SemaphoreType.DMA,
        ],
    )
    def sc_gather(u_hbm, v_hbm, emb_hbm, bh_hbm, bt_hbm,
                  h_out, t_out, bh_out, bt_out,
                  uidx_v, vidx_v, h_v, t_v, bh_v, bt_v, sem, sem2):
        wid = lax.axis_index("s") * _NC + lax.axis_index("c")
        base = wid * _BPW
        pltpu.sync_copy(u_hbm.at[pl.ds(base, _BPW)], uidx_v)
        pltpu.sync_copy(v_hbm.at[pl.ds(base, _BPW)], vidx_v)
        c1 = pltpu.async_copy(bh_hbm.at[uidx_v], bh_v, sem2)
        c2 = pltpu.async_copy(bt_hbm.at[vidx_v], bt_v, sem2)
        for c in range(_BPW // 16):
            vu = uidx_v[pl.ds(c * 16, 16)]
            vv = vidx_v[pl.ds(c * 16, 16)]
            for j in range(16):
                k = c * 16 + j
                pltpu.async_copy(
                    emb_hbm.at[pl.ds(vu[j], 1), :],
                    h_v.at[pl.ds(k, 1), :], sem)
                pltpu.async_copy(
                    emb_hbm.at[pl.ds(vv[j], 1), :],
                    t_v.at[pl.ds(k, 1), :], sem)
        pltpu.make_async_copy(
            emb_hbm.at[pl.ds(0, _BPW), :], h_v, sem).wait()
        pltpu.make_async_copy(
            emb_hbm.at[pl.ds(0, _BPW), :], t_v, sem).wait()
        c1.wait()
        c2.wait()
        pltpu.sync_copy(h_v, h_out.at[pl.ds(base, _BPW)])
        pltpu.sync_copy(t_v, t_out.at[pl.ds(base, _BPW)])
        pltpu.sync_copy(bh_v, bh_out.at[pl.ds(base, _BPW)])
        pltpu.sync_copy(bt_v, bt_out.at[pl.ds(base, _BPW)])

    return sc_gather


def _pair_consts():
    P = np.zeros((DIM, DIM), np.float32)
    E = np.zeros((DIM, DIM), np.float32)
    O = np.zeros((DIM, DIM), np.float32)
    for k in range(DIM // 2):
        P[2 * k + 1, 2 * k] = -1.0
        P[2 * k, 2 * k + 1] = 1.0
        E[2 * k, 2 * k] = 1.0
        E[2 * k, 2 * k + 1] = 1.0
        O[2 * k + 1, 2 * k] = 1.0
        O[2 * k + 1, 2 * k + 1] = 1.0
    return P, E, O


_P_MAT, _E_MAT, _O_MAT = _pair_consts()

_BM = 512


def _tc_body(scale_ref, h_ref, t_ref, r_ref,
             diag_ref, rbias_ref, bh_ref, bt_ref,
             pm_ref, em_ref, om_ref, o_ref):
    scale = scale_ref[0, 0]
    dot = functools.partial(
        lax.dot_general,
        dimension_numbers=(((1,), (0,)), ((), ())),
        preferred_element_type=jnp.float32,
    )

    h = h_ref[...]
    t = t_ref[...]

    rel = lax.broadcasted_iota(jnp.int32, (_BM, N_REL), 1)
    onehot = (rel == r_ref[...]).astype(jnp.float32)
    rd = dot(onehot, diag_ref[...])
    rb = dot(onehot, rbias_ref[...])

    a_bc = dot(rd, em_ref[...])
    b_bc = dot(rd, om_ref[...])
    inv_nrm = 1.0 / jnp.maximum(jnp.sqrt(a_bc * a_bc + b_bc * b_bc), 1e-15)
    h_sw = dot(h, pm_ref[...])
    x_rot = (a_bc * h + b_bc * h_sw) * inv_nrm

    col = lax.broadcasted_iota(jnp.int32, (_BM, DIM), 1)
    time = jax.nn.sigmoid(x_rot[:, 0:1]) * scale + 1.1
    x = x_rot + rb
    xn = jnp.where(col > 0, x, 0.0)
    s2 = jnp.sum(xn * xn, axis=1, keepdims=True)
    factor = jnp.sqrt((time * time - 1.0) / s2)
    h_l = jnp.where(col == 0, -time, x * factor)

    scores = lax.dot_general(
        h_l, t,
        dimension_numbers=(((1,), (1,)), ((), ())),
        preferred_element_type=jnp.float32,
    )
    o_ref[...] = MARGIN + 2.0 * scores + bh_ref[...] + bt_ref[...]


def kernel(u_idx, r_idx, v_idx, emb_entity, relation_bias, diag,
           bias_head, bias_tail, scale):
    u_idx = u_idx.astype(jnp.int32)
    v_idx = v_idx.astype(jnp.int32)
    r_idx = r_idx.astype(jnp.int32)

    h, t, bh_g, bt_g = _build_sc_gather()(
        u_idx, v_idx, emb_entity, bias_head, bias_tail)

    scale2 = scale.reshape(1, 1).astype(jnp.float32)
    r_col = r_idx.reshape(B, 1)
    bh_col = bh_g.reshape(B, 1)
    bt_row = bt_g.reshape(1, B)

    out = pl.pallas_call(
        _tc_body,
        grid=(B // _BM,),
        in_specs=[
            pl.BlockSpec((1, 1), lambda i: (0, 0), memory_space=pltpu.SMEM),
            pl.BlockSpec((_BM, DIM), lambda i: (i, 0)),
            pl.BlockSpec((B, DIM), lambda i: (0, 0)),
            pl.BlockSpec((_BM, 1), lambda i: (i, 0)),
            pl.BlockSpec((N_REL, DIM), lambda i: (0, 0)),
            pl.BlockSpec((N_REL, DIM), lambda i: (0, 0)),
            pl.BlockSpec((_BM, 1), lambda i: (i, 0)),
            pl.BlockSpec((1, B), lambda i: (0, 0)),
            pl.BlockSpec((DIM, DIM), lambda i: (0, 0)),
            pl.BlockSpec((DIM, DIM), lambda i: (0, 0)),
            pl.BlockSpec((DIM, DIM), lambda i: (0, 0)),
        ],
        out_specs=pl.BlockSpec((_BM, B), lambda i: (i, 0)),
        out_shape=jax.ShapeDtypeStruct((B, B), jnp.float32),
        compiler_params=pltpu.CompilerParams(
            dimension_semantics=("arbitrary",),
        ),
    )(scale2, h, t, r_col,
      diag, relation_bias, bh_col, bt_row,
      jnp.asarray(_P_MAT), jnp.asarray(_E_MAT), jnp.asarray(_O_MAT))
    return out
